# Initial kernel scaffold; baseline (speedup 1.0000x reference)
import jax
import jax.numpy as jnp
from jax.experimental import pallas as pl


def kernel(log_pa, score, v_label, v_l, role_label, roleset_id, log_frame, frame_idx, frame_pool):
    raise NotImplementedError("write your pallas kernel here")


# speedup vs baseline: 1.1466x; 1.1466x over previous
import functools

import jax
import jax.numpy as jnp
from jax import lax
from jax.experimental import pallas as pl
from jax.experimental.pallas import tpu as pltpu

B, L, NL, NF, NV = 16, 128, 40, 32, 20


def _body(vlab_ref, fidx_ref, vl_ref, lp_ref, pool_ref, lf_ref, out_ref):
    p = pl.program_id(0)
    i = p // NV
    v = p % NV

    @pl.when(p == 0)
    def _init():
        out_ref[...] = jnp.zeros((1, 1), jnp.float32)

    x = lp_ref[0, 0]
    xmax = jnp.max(x, axis=0, keepdims=True)
    w = jnp.maximum(1.0 - jnp.exp(xmax), 1e-6)
    pool = pool_ref[0]
    cand = jnp.where(pool == 0, w, 1.0)
    wm = jnp.min(cand, axis=1, keepdims=True)
    m = jnp.log(wm)
    fpred = lf_ref[0, 0].reshape(NF, 1)
    s = jnp.sum(jnp.maximum(fpred - m, 0.0))
    maskf = jnp.where(v < vl_ref[i], 1.0, 0.0)
    out_ref[...] += jnp.full((1, 1), maskf * s, jnp.float32)

    @pl.when(p == B * NV - 1)
    def _fini():
        tot = lax.fori_loop(0, B, lambda k, acc: acc + vl_ref[k], 0)
        norm = jnp.maximum(tot, 1).astype(jnp.float32)
        out_ref[...] = out_ref[...] / norm


@jax.jit
def _frame_role_loss(log_pa, v_label, v_l, log_frame, frame_idx, frame_pool):
    vlab = v_label.reshape(-1).astype(jnp.int32)
    fidx = jnp.take_along_axis(frame_idx, v_label, axis=1)
    fidx = fidx.reshape(-1).astype(jnp.int32)
    vl = v_l.astype(jnp.int32)

    grid_spec = pltpu.PrefetchScalarGridSpec(
        num_scalar_prefetch=3,
        grid=(B * NV,),
        in_specs=[
            pl.BlockSpec((1, 1, L, NL),
                         lambda p, vlab, fidx, vl: (p // NV, vlab[p], 0, 0)),
            pl.BlockSpec((1, NF, NL),
                         lambda p, vlab, fidx, vl: (fidx[p], 0, 0)),
            pl.BlockSpec((1, 1, 1, NF),
                         lambda p, vlab, fidx, vl: (p // NV, vlab[p], 0, 0)),
        ],
        out_specs=pl.BlockSpec((1, 1), lambda p, vlab, fidx, vl: (0, 0)),
    )
    out = pl.pallas_call(
        _body,
        grid_spec=grid_spec,
        out_shape=jax.ShapeDtypeStruct((1, 1), jnp.float32),
    )(vlab, fidx, vl, log_pa, frame_pool,
      log_frame.reshape(B, L, 1, NF))
    return out.reshape(())


def kernel(log_pa, score, v_label, v_l, role_label, roleset_id, log_frame,
           frame_idx, frame_pool):
    return _frame_role_loss(log_pa, v_label, v_l, log_frame, frame_idx,
                            frame_pool)


# speedup vs baseline: 1.8011x; 1.5708x over previous
import functools

import jax
import jax.numpy as jnp
from jax import lax
from jax.experimental import pallas as pl
from jax.experimental.pallas import tpu as pltpu
from jax.experimental.pallas import tpu_sc as plsc

B, L, NL, NF, NV = 16, 128, 40, 32, 20
NW = 32
RPW = (B * NV) // NW
ROW = L * NL
PROW = NF * NL
NEG = -3.0e38

_mesh = plsc.VectorSubcoreMesh(core_axis_name="c", subcore_axis_name="s")


@functools.partial(
    pl.kernel,
    out_type=(jax.ShapeDtypeStruct((NW, RPW, NF, 16), jnp.float32),
              jax.ShapeDtypeStruct((NW, RPW, 128), jnp.float32)),
    mesh=_mesh,
    compiler_params=pltpu.CompilerParams(use_tc_tiling_on_sc=False),
    scratch_types=[
        pltpu.VMEM((NW, RPW), jnp.int32),
        pltpu.VMEM((NW, RPW), jnp.int32),
        pltpu.VMEM((NW, RPW), jnp.int32),
        pltpu.VMEM((RPW, ROW), jnp.float32),
        pltpu.VMEM((RPW, PROW), jnp.int32),
        pltpu.VMEM((RPW, 128), jnp.float32),
        pltpu.VMEM((RPW, NF, 16), jnp.float32),
        pltpu.VMEM((80,), jnp.float32),
        pltpu.SemaphoreType.DMA,
        pltpu.SemaphoreType.DMA,
        pltpu.SemaphoreType.DMA,
    ],
)
def _sc_gather_reduce(lp_hbm, pool_hbm, lf_hbm, ridx_hbm, fidx_hbm, sidx_hbm,
                      wm_out, fp_out,
                      idx_all, fidx_all, sidx_all, rows_v, pool_v, fp_v,
                      wm_v, s80, sem0, sem1, sem2):
    wid = lax.axis_index("s") * 2 + lax.axis_index("c")
    pltpu.sync_copy(ridx_hbm, idx_all)
    pltpu.sync_copy(fidx_hbm, fidx_all)
    pltpu.sync_copy(sidx_hbm, sidx_all)
    cp0 = pltpu.async_copy(lp_hbm.at[idx_all.at[wid]], rows_v, sem0)
    cp1 = pltpu.async_copy(pool_hbm.at[fidx_all.at[wid]], pool_v, sem1)
    cp2 = pltpu.async_copy(lf_hbm.at[sidx_all.at[wid]], fp_v, sem2)
    cp0.wait()
    cp1.wait()
    cp2.wait()

    def row_body(j, carry):
        def g_body(g, accs):
            base = pl.multiple_of(g * 80, 16)
            return tuple(
                jnp.maximum(a, rows_v[j, pl.ds(base + 16 * p, 16)])
                for p, a in enumerate(accs)
            )

        init = tuple(jnp.full((16,), NEG, jnp.float32) for _ in range(5))
        accs = lax.fori_loop(0, ROW // 80, g_body, init)
        for p in range(5):
            s80[pl.ds(16 * p, 16)] = accs[p]
        f0 = jnp.maximum(s80[pl.ds(0, 16)], s80[pl.ds(40, 16)])
        f1 = jnp.maximum(s80[pl.ds(16, 16)], s80[pl.ds(56, 16)])
        f2 = jnp.maximum(s80[pl.ds(24, 16)], s80[pl.ds(64, 16)])
        w0 = jnp.maximum(1.0 - jnp.exp(f0), 1e-6)
        w1 = jnp.maximum(1.0 - jnp.exp(f1), 1e-6)
        w2 = jnp.maximum(1.0 - jnp.exp(f2), 1e-6)
        for f in range(NF):
            off = f * NL
            c0 = jnp.where(pool_v[j, pl.ds(off, 16)] == 0, w0, 1.0)
            c1 = jnp.where(pool_v[j, pl.ds(off + 16, 16)] == 0, w1, 1.0)
            c2 = jnp.where(pool_v[j, pl.ds(off + 24, 16)] == 0, w2, 1.0)
            wm_v[j, f] = jnp.minimum(c0, jnp.minimum(c1, c2))
        return carry

    lax.fori_loop(0, RPW, row_body, 0)
    pltpu.sync_copy(wm_v, wm_out.at[wid])
    pltpu.sync_copy(fp_v, fp_out.at[wid])


def _tc_body(wm_ref, fp_ref, qsel_ref, vl_ref, out_ref):
    wm = jnp.min(wm_ref[...], axis=3)
    raw = fp_ref[...]
    q = qsel_ref[...]
    fp = (jnp.where(q == 0, raw[:, :, 0:NF], 0.0)
          + jnp.where(q == 1, raw[:, :, NF:2 * NF], 0.0)
          + jnp.where(q == 2, raw[:, :, 2 * NF:3 * NF], 0.0)
          + jnp.where(q == 3, raw[:, :, 3 * NF:], 0.0))
    m = jnp.log(wm)
    t = jnp.maximum(fp - m, 0.0)
    s = jnp.sum(t, axis=2)
    iota_v = lax.broadcasted_iota(jnp.int32, (B, NV), 1)
    vl = vl_ref[...]
    masked = jnp.where(iota_v < vl, s, 0.0)
    norm = jnp.maximum(jnp.sum(vl), 1).astype(jnp.float32)
    out_ref[...] = jnp.full((1, 1), jnp.sum(masked) / norm, jnp.float32)


@jax.jit
def _frame_role_loss(log_pa, v_label, v_l, log_frame, frame_idx, frame_pool):
    lp_flat = log_pa.reshape(B * L, ROW)
    pool_flat = frame_pool.reshape(-1, PROW).astype(jnp.int32)
    lf_flat = log_frame.reshape((B * L * NF) // 128, 128)
    vlab = v_label.astype(jnp.int32)
    ridx = (jnp.arange(B, dtype=jnp.int32)[:, None] * L + vlab)
    sidx = ridx // 4
    qsel = (ridx % 4).reshape(B, NV, 1)
    fidx = jnp.take_along_axis(frame_idx.astype(jnp.int32), vlab, axis=1)
    wm, fpred = _sc_gather_reduce(
        lp_flat, pool_flat, lf_flat,
        ridx.reshape(NW, RPW), fidx.reshape(NW, RPW), sidx.reshape(NW, RPW))
    loss = pl.pallas_call(
        _tc_body,
        out_shape=jax.ShapeDtypeStruct((1, 1), jnp.float32),
    )(wm.reshape(B, NV, NF, 16), fpred.reshape(B, NV, 128), qsel,
      v_l.reshape(B, 1).astype(jnp.int32))
    return loss.reshape(())


def kernel(log_pa, score, v_label, v_l, role_label, roleset_id, log_frame,
           frame_idx, frame_pool):
    return _frame_role_loss(log_pa, v_label, v_l, log_frame, frame_idx,
                            frame_pool)


# speedup vs baseline: 2.3396x; 1.2990x over previous
import functools

import jax
import jax.numpy as jnp
from jax import lax
from jax.experimental import pallas as pl
from jax.experimental.pallas import tpu as pltpu
from jax.experimental.pallas import tpu_sc as plsc

B, L, NL, NF, NV = 16, 128, 40, 32, 20
NW = 32
RPW = (B * NV) // NW
ROW = L * NL
PROW = NF * NL
NEG = -3.0e38

_mesh = plsc.VectorSubcoreMesh(core_axis_name="c", subcore_axis_name="s")


def _pool_plan():
    plans = []
    for f in range(NF):
        a = f * NL
        sl = []
        for s, wo in ((a, 8), (a + 16, 24), (a + 24, 32)):
            if s % 128 == 120:
                sl.append((s - 8, wo - 8))
                sl.append((s + 8, wo + 8))
            else:
                sl.append((s, wo))
        plans.append(sl)
    return plans


_POOL_PLANS = _pool_plan()


@functools.partial(
    pl.kernel,
    out_type=(jax.ShapeDtypeStruct((NW, RPW, 4, 128), jnp.float32),
              jax.ShapeDtypeStruct((NW, RPW, 128), jnp.float32)),
    mesh=_mesh,
    scratch_types=[
        pltpu.VMEM((NW, RPW), jnp.int32),
        pltpu.VMEM((NW, RPW), jnp.int32),
        pltpu.VMEM((NW, RPW), jnp.int32),
        pltpu.VMEM((8, ROW), jnp.float32),
        pltpu.VMEM((2, ROW), jnp.float32),
        pltpu.VMEM((8, PROW), jnp.int32),
        pltpu.VMEM((2, PROW), jnp.int32),
        pltpu.VMEM((RPW, 128), jnp.float32),
        pltpu.VMEM((RPW, 4, 128), jnp.float32),
        pltpu.VMEM((80,), jnp.float32),
        pltpu.VMEM((64,), jnp.float32),
        pltpu.SemaphoreType.DMA,
        pltpu.SemaphoreType.DMA,
        pltpu.SemaphoreType.DMA,
    ],
)
def _sc_gather_reduce(lp_hbm, pool_hbm, lf_hbm, ridx_hbm, fidx_hbm, sidx_hbm,
                      wm_out, fp_out,
                      idx_all, fidx_all, sidx_all, rows_a, rows_b, pool_a,
                      pool_b, fp_v, wm_v, s80, wbuf, sem0, sem1, sem2):
    wid = lax.axis_index("s") * 2 + lax.axis_index("c")
    pltpu.sync_copy(ridx_hbm, idx_all)
    pltpu.sync_copy(fidx_hbm, fidx_all)
    pltpu.sync_copy(sidx_hbm, sidx_all)
    cp0 = pltpu.async_copy(lp_hbm.at[idx_all.at[wid, pl.ds(0, 8)]],
                           rows_a, sem0)
    cp1 = pltpu.async_copy(lp_hbm.at[idx_all.at[wid, pl.ds(8, 2)]],
                           rows_b, sem0)
    cp2 = pltpu.async_copy(pool_hbm.at[fidx_all.at[wid, pl.ds(0, 8)]],
                           pool_a, sem1)
    cp3 = pltpu.async_copy(pool_hbm.at[fidx_all.at[wid, pl.ds(8, 2)]],
                           pool_b, sem1)
    cp4 = pltpu.async_copy(lf_hbm.at[sidx_all.at[wid]], fp_v, sem2)
    cp0.wait()
    cp1.wait()
    cp2.wait()
    cp3.wait()
    cp4.wait()

    def make_row_body(rows_v, pool_v, j0):
        def row_body(j, carry):
            def g_body(g, accs):
                base = pl.multiple_of(g * 80, 16)
                return tuple(
                    jnp.maximum(a, rows_v[j, pl.ds(base + 16 * p, 16)])
                    for p, a in enumerate(accs)
                )

            init = tuple(jnp.full((16,), NEG, jnp.float32) for _ in range(5))
            accs = lax.fori_loop(0, ROW // 80, g_body, init)
            for p in range(5):
                s80[pl.ds(16 * p, 16)] = accs[p]
            f0 = jnp.maximum(s80[pl.ds(0, 16)], s80[pl.ds(40, 16)])
            f1 = jnp.maximum(s80[pl.ds(16, 16)], s80[pl.ds(56, 16)])
            f2 = jnp.maximum(s80[pl.ds(24, 16)], s80[pl.ds(64, 16)])
            ones = jnp.ones((16,), jnp.float32)
            wbuf[pl.ds(0, 16)] = ones
            wbuf[pl.ds(48, 16)] = ones
            wbuf[pl.ds(8, 16)] = jnp.maximum(1.0 - jnp.exp(f0), 1e-6)
            wbuf[pl.ds(24, 16)] = jnp.maximum(1.0 - jnp.exp(f1), 1e-6)
            wbuf[pl.ds(32, 16)] = jnp.maximum(1.0 - jnp.exp(f2), 1e-6)
            for f in range(NF):
                c = None
                for s, wo in _POOL_PLANS[f]:
                    wk = wbuf[pl.ds(wo, 16)]
                    ck = jnp.where(pool_v[j, pl.ds(s, 16)] == 0, wk, 1.0)
                    c = ck if c is None else jnp.minimum(c, ck)
                wm_v[j0 + j, f // 8, pl.ds((f % 8) * 16, 16)] = c
            return carry
        return row_body

    lax.fori_loop(0, 8, make_row_body(rows_a, pool_a, 0), 0)
    lax.fori_loop(0, 2, make_row_body(rows_b, pool_b, 8), 0)
    pltpu.sync_copy(wm_v, wm_out.at[wid])
    pltpu.sync_copy(fp_v, fp_out.at[wid])


def _tc_body(wm_ref, fp_ref, qsel_ref, vl_ref, out_ref):
    wm = jnp.min(wm_ref[...], axis=3)
    raw = fp_ref[...]
    q = qsel_ref[...]
    fp = (jnp.where(q == 0, raw[:, :, 0:NF], 0.0)
          + jnp.where(q == 1, raw[:, :, NF:2 * NF], 0.0)
          + jnp.where(q == 2, raw[:, :, 2 * NF:3 * NF], 0.0)
          + jnp.where(q == 3, raw[:, :, 3 * NF:], 0.0))
    m = jnp.log(wm)
    t = jnp.maximum(fp - m, 0.0)
    s = jnp.sum(t, axis=2)
    iota_v = lax.broadcasted_iota(jnp.int32, (B, NV), 1)
    vl = vl_ref[...]
    masked = jnp.where(iota_v < vl, s, 0.0)
    norm = jnp.maximum(jnp.sum(vl), 1).astype(jnp.float32)
    out_ref[...] = jnp.full((1, 1), jnp.sum(masked) / norm, jnp.float32)


@jax.jit
def _frame_role_loss(log_pa, v_label, v_l, log_frame, frame_idx, frame_pool):
    lp_flat = log_pa.reshape(B * L, ROW)
    pool_flat = frame_pool.reshape(-1, PROW).astype(jnp.int32)
    lf_flat = log_frame.reshape((B * L * NF) // 128, 128)
    vlab = v_label.astype(jnp.int32)
    ridx = (jnp.arange(B, dtype=jnp.int32)[:, None] * L + vlab)
    sidx = ridx // 4
    qsel = (ridx % 4).reshape(B, NV, 1)
    fidx = jnp.take_along_axis(frame_idx.astype(jnp.int32), vlab, axis=1)
    wm, fpred = _sc_gather_reduce(
        lp_flat, pool_flat, lf_flat,
        ridx.reshape(NW, RPW), fidx.reshape(NW, RPW), sidx.reshape(NW, RPW))
    loss = pl.pallas_call(
        _tc_body,
        out_shape=jax.ShapeDtypeStruct((1, 1), jnp.float32),
    )(wm.reshape(B, NV, NF, 16), fpred.reshape(B, NV, 128), qsel,
      v_l.reshape(B, 1).astype(jnp.int32))
    return loss.reshape(())


def kernel(log_pa, score, v_label, v_l, role_label, roleset_id, log_frame,
           frame_idx, frame_pool):
    return _frame_role_loss(log_pa, v_label, v_l, log_frame, frame_idx,
                            frame_pool)
